# baseline (speedup 1.0000x reference)
import jax
import jax.numpy as jnp
from jax import lax
from jax.experimental import pallas as pl
from jax.experimental.pallas import tpu as pltpu
from jax.experimental.pallas import tpu_sc as plsc

D = 64
V = 1000000
NC = 2
NS = 16
NW = NC * NS
NF = 26
BATCH = 16384
CH = 128
NROWS = NF * BATCH // CH
RPW = NROWS // NW
NB = 4
NCHUNKS = V // CH
VTAIL = V - NCHUNKS * CH

_MESH = dict(
    mesh=plsc.VectorSubcoreMesh(core_axis_name="c", subcore_axis_name="s"),
    compiler_params=pltpu.CompilerParams(needs_layout_passes=False),
)


def _repack_body(wt_hbm, wtail_hbm, w128_hbm, in_v, tr_v, rsem, wsem):
    wid = lax.axis_index("s") * NC + lax.axis_index("c")
    iota = lax.iota(jnp.int32, 16)

    def chunk_cols(ci):
        return pl.ds(pl.multiple_of(ci * CH, CH), CH)

    def transpose(src, dst, width):
        @pl.loop(0, width // 16)
        def _(jb):
            jvec = iota + 16 * jb

            @pl.loop(0, D, unroll=8)
            def _(dd):
                vals = src[dd, pl.ds(jb * 16, 16)]
                plsc.store_scatter(dst, [jvec, jnp.full((16,), dd, jnp.int32)], vals)

    pltpu.async_copy(wt_hbm.at[:, chunk_cols(wid)], in_v.at[0], rsem.at[0])
    pltpu.async_copy(wt_hbm.at[:, chunk_cols(wid + NW)], in_v.at[1], rsem.at[1])

    @pl.loop(wid, NCHUNKS, step=NW)
    def _(ci):
        s = ((ci - wid) // NW) & 1
        pltpu.make_async_copy(
            wt_hbm.at[:, pl.ds(0, CH)], in_v.at[s], rsem.at[s]).wait()

        @pl.when(ci >= wid + 2 * NW)
        def _():
            pltpu.make_async_copy(
                tr_v.at[s], w128_hbm.at[pl.ds(0, CH)], wsem.at[s]).wait()

        transpose(in_v.at[s], tr_v.at[s], CH)
        pltpu.async_copy(tr_v.at[s], w128_hbm.at[chunk_cols(ci)], wsem.at[s])

        @pl.when(ci + 2 * NW < NCHUNKS)
        def _():
            pltpu.async_copy(
                wt_hbm.at[:, chunk_cols(ci + 2 * NW)], in_v.at[s], rsem.at[s])

    for s in range(2):
        pltpu.make_async_copy(
            tr_v.at[s], w128_hbm.at[pl.ds(0, CH)], wsem.at[s]).wait()

    @pl.when(wid == 0)
    def _():
        pltpu.sync_copy(wtail_hbm, w128_hbm.at[pl.ds(V - VTAIL, VTAIL)])


def _transpose_slab(rows, tr):
    iota = lax.iota(jnp.int32, 16)

    @pl.loop(0, CH // 16)
    def _(jb):
        rvec = iota + 16 * jb

        @pl.loop(0, D, unroll=8)
        def _(dd):
            vals = plsc.load_gather(rows, [rvec, jnp.full((16,), dd, jnp.int32)])
            tr[dd, pl.ds(jb * 16, 16)] = vals


def _gather_body(idx_hbm, table_hbm, out_hbm, idx_v, rows_v, tr_v, gsem, osem):
    wid = lax.axis_index("s") * NC + lax.axis_index("c")
    r0 = RPW * wid
    pltpu.sync_copy(idx_hbm.at[pl.ds(pl.multiple_of(r0, 8), RPW)], idx_v)

    def gather(u, s):
        pltpu.async_copy(table_hbm.at[idx_v.at[u]], rows_v.at[s], gsem.at[s])

    def wait_gather(s):
        pltpu.make_async_copy(
            table_hbm.at[idx_v.at[0]], rows_v.at[s], gsem.at[s]).wait()

    def put(u, s):
        r = r0 + u
        f, c = r // CH, r % CH
        dst = out_hbm.at[f, :, pl.ds(pl.multiple_of(CH * c, CH), CH)]
        pltpu.async_copy(tr_v.at[s], dst, osem.at[s])

    def wait_put(s):
        pltpu.make_async_copy(
            tr_v.at[s], out_hbm.at[0, :, pl.ds(0, CH)], osem.at[s]).wait()

    for s in range(NB):
        gather(s, s)
    for s in range(NB):
        wait_gather(s)
        _transpose_slab(rows_v.at[s], tr_v.at[s])
        put(s, s)
        gather(s + NB, s)

    @pl.loop(NB, RPW - NB, step=NB)
    def _(u0):
        for s in range(NB):
            u = u0 + s
            wait_gather(s)
            wait_put(s)
            _transpose_slab(rows_v.at[s], tr_v.at[s])
            put(u, s)
            gather(u + NB, s)

    for s in range(NB):
        u = RPW - NB + s
        wait_gather(s)
        wait_put(s)
        _transpose_slab(rows_v.at[s], tr_v.at[s])
        put(u, s)
    for s in range(NB):
        wait_put(s)


def kernel(idx, weight):
    wt = weight.T
    wtail = jnp.pad(weight[V - VTAIL:, :], ((0, 0), (0, 128 - D)))
    idxr = idx.T.reshape(NROWS, CH).astype(jnp.int32)
    repack = pl.kernel(
        _repack_body,
        out_type=jax.ShapeDtypeStruct((V, 128), jnp.float32),
        scratch_types=[
            pltpu.VMEM((2, D, CH), jnp.float32),
            pltpu.VMEM((2, CH, 128), jnp.float32),
            pltpu.SemaphoreType.DMA((2,)),
            pltpu.SemaphoreType.DMA((2,)),
        ],
        **_MESH,
    )
    gather = pl.kernel(
        _gather_body,
        out_type=jax.ShapeDtypeStruct((NF, D, BATCH), jnp.float32),
        scratch_types=[
            pltpu.VMEM((RPW, CH), jnp.int32),
            pltpu.VMEM((NB, CH, 128), jnp.float32),
            pltpu.VMEM((NB, D, CH), jnp.float32),
            pltpu.SemaphoreType.DMA((NB,)),
            pltpu.SemaphoreType.DMA((NB,)),
        ],
        **_MESH,
    )
    out_t = gather(idxr, repack(wt, wtail))
    return out_t.transpose(2, 0, 1)


# speedup vs baseline: 2.4309x; 2.4309x over previous
import jax
import jax.numpy as jnp
from jax import lax
from jax.experimental import pallas as pl
from jax.experimental.pallas import tpu as pltpu
from jax.experimental.pallas import tpu_sc as plsc

D = 64
V = 1000000
NC = 2
NS = 16
NW = NC * NS
NF = 26
BATCH = 16384
CH = 128
NROWS = NF * BATCH // CH
RPW = NROWS // NW
NB = 4

_MESH = dict(
    mesh=plsc.VectorSubcoreMesh(core_axis_name="c", subcore_axis_name="s"),
    compiler_params=pltpu.CompilerParams(needs_layout_passes=False),
)


def _gather_body(idx_hbm, table_hbm, out_hbm, idx_v, rows_v, gsem, osem):
    wid = lax.axis_index("s") * NC + lax.axis_index("c")
    r0 = RPW * wid
    pltpu.sync_copy(idx_hbm.at[pl.ds(pl.multiple_of(r0, 8), RPW)], idx_v)

    def gather(u, s):
        pltpu.async_copy(table_hbm.at[idx_v.at[u]], rows_v.at[s], gsem.at[s])

    def wait_gather(s):
        pltpu.make_async_copy(
            table_hbm.at[idx_v.at[0]], rows_v.at[s], gsem.at[s]).wait()

    def put(u, s):
        pltpu.async_copy(rows_v.at[s], out_hbm.at[r0 + u], osem.at[s])

    def wait_put(s):
        pltpu.make_async_copy(rows_v.at[s], out_hbm.at[0], osem.at[s]).wait()

    for s in range(NB):
        gather(s, s)

    @pl.loop(0, RPW - NB, step=NB)
    def _(u0):
        for s in range(NB):
            u = u0 + s
            wait_gather(s)
            put(u, s)
            wait_put(s)
            gather(u + NB, s)

    for s in range(NB):
        u = RPW - NB + s
        wait_gather(s)
        put(u, s)
    for s in range(NB):
        wait_put(s)


def kernel(idx, weight):
    w128 = jnp.pad(weight, ((0, 0), (0, 128 - D)))
    idxr = idx.T.reshape(NROWS, CH).astype(jnp.int32)
    gather = pl.kernel(
        _gather_body,
        out_type=jax.ShapeDtypeStruct((NROWS, CH, 128), jnp.float32),
        scratch_types=[
            pltpu.VMEM((RPW, CH), jnp.int32),
            pltpu.VMEM((NB, CH, 128), jnp.float32),
            pltpu.SemaphoreType.DMA((NB,)),
            pltpu.SemaphoreType.DMA((NB,)),
        ],
        **_MESH,
    )
    out3 = gather(idxr, w128)
    out = out3.reshape(NF, BATCH, 128)[:, :, :D]
    return out.transpose(1, 0, 2)
